# baseline (speedup 1.0000x reference)
import jax
import jax.numpy as jnp
from jax import lax
from jax.experimental import pallas as pl
from jax.experimental.pallas import tpu as pltpu
from jax.experimental.pallas import tpu_sc as plsc

N = 8192
D = 64
B = 4096

_NC = 2
_NS = 16
_NW = _NC * _NS
_PB = B // _NW
_L = 16
_NG = _PB // _L

_LN2 = 0.6931471805599453
_LN100 = 4.605170185988092


def _sc_body(inp_hbm, outp_hbm, co_hbm, ie_hbm, ib_hbm, oe_hbm, ob_hbm,
             parts_out,
             inp_v, outp_v, flat_v, x_v, ib_v, ob_v,
             ie_v, oe_v, part_v, sem):
    wid = lax.axis_index("s") * _NC + lax.axis_index("c")
    base = wid * _PB
    idx_cp1 = pltpu.async_copy(inp_hbm.at[pl.ds(base, _PB)], inp_v, sem)
    idx_cp2 = pltpu.async_copy(outp_hbm.at[pl.ds(base, _PB)], outp_v, sem)
    idx_cp1.wait()
    idx_cp2.wait()
    for j in range(_NG):
        s = pl.ds(j * _L, _L)
        r = inp_v[s]
        c = outp_v[s]
        flat_v[s] = ((r >> 3) * 64 + (c >> 7)) * 1024 + (r & 7) * 128 + (c & 127)
    co_cp = pltpu.async_copy(co_hbm.at[flat_v], x_v, sem)
    emb_cp1 = pltpu.async_copy(ie_hbm.at[inp_v], ie_v, sem)
    emb_cp2 = pltpu.async_copy(oe_hbm.at[outp_v], oe_v, sem)
    rest = [
        co_cp,
        pltpu.async_copy(ib_hbm.at[inp_v], ib_v, sem),
        pltpu.async_copy(ob_hbm.at[outp_v], ob_v, sem),
    ]
    emb_cp1.wait()
    emb_cp2.wait()

    lanes = lax.iota(jnp.int32, _L)
    rows = [g * _L + lanes for g in range(_NG)]
    zero = jnp.zeros((_L,), jnp.float32)

    def dot_step(k, accs):
        col = (k + lanes) & (D - 1)
        return tuple(
            accs[g] + plsc.load_gather(ie_v, [rows[g], col])
            * plsc.load_gather(oe_v, [rows[g], col])
            for g in range(_NG))

    dots = lax.fori_loop(0, D, dot_step, tuple(zero for _ in range(_NG)),
                         unroll=8)

    for cp in rest:
        cp.wait()

    s1 = s2 = s3 = c1 = c2 = zero
    for g in range(_NG):
        s = pl.ds(g * _L, _L)
        xv = x_v[s] + 1.0
        bits = plsc.bitcast(xv, jnp.int32)
        e = (bits >> 23) - 127
        m = plsc.bitcast((bits & 0x007FFFFF) | 0x3F800000, jnp.float32)
        t = (m - 1.0) / (m + 1.0)
        t2 = t * t
        lnm = 2.0 * t * (1.0 + t2 * (1.0 / 3 + t2 * (1.0 / 5 + t2 * (1.0 / 7 + t2 * (1.0 / 9)))))
        lnx = e.astype(jnp.float32) * _LN2 + lnm
        d = dots[g] - lnx
        w = jnp.where(xv > 100.0, 1.0, jnp.exp(0.75 * (lnx - _LN100)))
        wd = w * d
        s1 = s1 + wd * d
        s2 = s2 + wd
        s3 = s3 + w
        cv = ib_v[s] + ob_v[s]
        c1 = c1 + cv
        c2 = c2 + cv * cv

    for k in range(8):
        part_v[pl.ds(k * _L, _L)] = zero
    for k, v in enumerate((s1, s2, s3, c1, c2)):
        part_v[pl.ds(k * _L, _L)] = v
    pltpu.sync_copy(part_v, parts_out.at[wid])


_sc_glove = pl.kernel(
    _sc_body,
    out_type=jax.ShapeDtypeStruct((_NW, 128), jnp.float32),
    mesh=plsc.VectorSubcoreMesh(core_axis_name="c", subcore_axis_name="s"),
    scratch_types=[
        pltpu.VMEM((_PB,), jnp.int32),
        pltpu.VMEM((_PB,), jnp.int32),
        pltpu.VMEM((_PB,), jnp.int32),
        pltpu.VMEM((_PB,), jnp.float32),
        pltpu.VMEM((_PB,), jnp.float32),
        pltpu.VMEM((_PB,), jnp.float32),
        pltpu.VMEM((_PB, D), jnp.float32),
        pltpu.VMEM((_PB, D), jnp.float32),
        pltpu.VMEM((128,), jnp.float32),
        pltpu.SemaphoreType.DMA,
    ],
    compiler_params=pltpu.CompilerParams(use_tc_tiling_on_sc=False,
                                         needs_layout_passes=False),
)


def _tc_body(parts_ref, out_ref):
    p = parts_ref[:]
    s1 = jnp.sum(p[:, 0:16])
    s2 = jnp.sum(p[:, 16:32])
    s3 = jnp.sum(p[:, 32:48])
    c1 = jnp.sum(p[:, 48:64])
    c2 = jnp.sum(p[:, 64:80])
    out_ref[0, 0] = B * s1 + 2.0 * c1 * s2 + c2 * s3


_tc_combine = pl.pallas_call(
    _tc_body,
    out_shape=jax.ShapeDtypeStruct((1, 1), jnp.float32),
    out_specs=pl.BlockSpec(memory_space=pltpu.SMEM),
)


def kernel(input, output, co_occur, in_embed, in_bias, out_embed, out_bias):
    co_flat = (co_occur.reshape(N // 8, 8, N // 128, 128)
               .transpose(0, 2, 1, 3).reshape(N * N))
    parts = _sc_glove(
        input.astype(jnp.int32), output.astype(jnp.int32), co_flat,
        in_embed, in_bias.reshape(N), out_embed, out_bias.reshape(N))
    res = _tc_combine(parts)
    return res[0, 0]


# speedup vs baseline: 1.0668x; 1.0668x over previous
import jax
import jax.numpy as jnp
from jax import lax
from jax.experimental import pallas as pl
from jax.experimental.pallas import tpu as pltpu
from jax.experimental.pallas import tpu_sc as plsc

N = 8192
D = 64
B = 4096

_NC = 2
_NS = 16
_NW = _NC * _NS
_PB = B // _NW
_L = 16
_NG = _PB // _L

_LN2 = 0.6931471805599453
_LN100 = 4.605170185988092


def _sc_body(inp_hbm, outp_hbm, co_hbm, ie_hbm, ib_hbm, oe_hbm, ob_hbm,
             parts_out,
             inp_v, outp_v, flat_v, x_v, ib_v, ob_v,
             ie_v, oe_v, part_v, sem):
    wid = lax.axis_index("s") * _NC + lax.axis_index("c")
    base = wid * _PB
    idx_cp1 = pltpu.async_copy(inp_hbm.at[pl.ds(base, _PB)], inp_v, sem)
    idx_cp2 = pltpu.async_copy(outp_hbm.at[pl.ds(base, _PB)], outp_v, sem)
    idx_cp1.wait()
    idx_cp2.wait()
    for j in range(_NG):
        s = pl.ds(j * _L, _L)
        r = inp_v[s]
        c = outp_v[s]
        flat_v[s] = ((r >> 3) * 64 + (c >> 7)) * 1024 + (r & 7) * 128 + (c & 127)
    co_cp = pltpu.async_copy(co_hbm.at[flat_v], x_v, sem)
    emb_cp1 = pltpu.async_copy(ie_hbm.at[inp_v], ie_v, sem)
    emb_cp2 = pltpu.async_copy(oe_hbm.at[outp_v], oe_v, sem)
    rest = [
        co_cp,
        pltpu.async_copy(ib_hbm.at[inp_v], ib_v, sem),
        pltpu.async_copy(ob_hbm.at[outp_v], ob_v, sem),
    ]
    emb_cp1.wait()
    emb_cp2.wait()

    lanes = lax.iota(jnp.int32, _L)
    rows = [g * _L + lanes for g in range(_NG)]
    zero = jnp.zeros((_L,), jnp.float32)

    def dot_step(k, accs):
        col = (k + lanes) & (D - 1)
        return tuple(
            accs[g] + plsc.load_gather(ie_v, [rows[g], col])
            * plsc.load_gather(oe_v, [rows[g], col])
            for g in range(_NG))

    dots = lax.fori_loop(0, D, dot_step, tuple(zero for _ in range(_NG)),
                         unroll=2)

    for cp in rest:
        cp.wait()

    s1 = s2 = s3 = c1 = c2 = zero
    for g in range(_NG):
        s = pl.ds(g * _L, _L)
        xv = x_v[s] + 1.0
        bits = plsc.bitcast(xv, jnp.int32)
        e = (bits >> 23) - 127
        m = plsc.bitcast((bits & 0x007FFFFF) | 0x3F800000, jnp.float32)
        t = (m - 1.0) / (m + 1.0)
        t2 = t * t
        lnm = 2.0 * t * (1.0 + t2 * (1.0 / 3 + t2 * (1.0 / 5 + t2 * (1.0 / 7 + t2 * (1.0 / 9)))))
        lnx = e.astype(jnp.float32) * _LN2 + lnm
        d = dots[g] - lnx
        w = jnp.where(xv > 100.0, 1.0, jnp.exp(0.75 * (lnx - _LN100)))
        wd = w * d
        s1 = s1 + wd * d
        s2 = s2 + wd
        s3 = s3 + w
        cv = ib_v[s] + ob_v[s]
        c1 = c1 + cv
        c2 = c2 + cv * cv

    for k in range(8):
        part_v[pl.ds(k * _L, _L)] = zero
    for k, v in enumerate((s1, s2, s3, c1, c2)):
        part_v[pl.ds(k * _L, _L)] = v
    pltpu.sync_copy(part_v, parts_out.at[wid])


_sc_glove = pl.kernel(
    _sc_body,
    out_type=jax.ShapeDtypeStruct((_NW, 128), jnp.float32),
    mesh=plsc.VectorSubcoreMesh(core_axis_name="c", subcore_axis_name="s"),
    scratch_types=[
        pltpu.VMEM((_PB,), jnp.int32),
        pltpu.VMEM((_PB,), jnp.int32),
        pltpu.VMEM((_PB,), jnp.int32),
        pltpu.VMEM((_PB,), jnp.float32),
        pltpu.VMEM((_PB,), jnp.float32),
        pltpu.VMEM((_PB,), jnp.float32),
        pltpu.VMEM((_PB, D), jnp.float32),
        pltpu.VMEM((_PB, D), jnp.float32),
        pltpu.VMEM((128,), jnp.float32),
        pltpu.SemaphoreType.DMA,
    ],
    compiler_params=pltpu.CompilerParams(use_tc_tiling_on_sc=False,
                                         needs_layout_passes=False),
)


def _tc_body(parts_ref, out_ref):
    p = parts_ref[:]
    s1 = jnp.sum(p[:, 0:16])
    s2 = jnp.sum(p[:, 16:32])
    s3 = jnp.sum(p[:, 32:48])
    c1 = jnp.sum(p[:, 48:64])
    c2 = jnp.sum(p[:, 64:80])
    out_ref[0, 0] = B * s1 + 2.0 * c1 * s2 + c2 * s3


_tc_combine = pl.pallas_call(
    _tc_body,
    out_shape=jax.ShapeDtypeStruct((1, 1), jnp.float32),
    out_specs=pl.BlockSpec(memory_space=pltpu.SMEM),
)


def kernel(input, output, co_occur, in_embed, in_bias, out_embed, out_bias):
    co_flat = (co_occur.reshape(N // 8, 8, N // 128, 128)
               .transpose(0, 2, 1, 3).reshape(N * N))
    parts = _sc_glove(
        input.astype(jnp.int32), output.astype(jnp.int32), co_flat,
        in_embed, in_bias.reshape(N), out_embed, out_bias.reshape(N))
    res = _tc_combine(parts)
    return res[0, 0]
